# baseline (speedup 1.0000x reference)
import functools

import jax
import jax.numpy as jnp
from jax import lax
from jax.experimental import pallas as pl
from jax.experimental.pallas import tpu as pltpu
from jax.experimental.pallas import tpu_sc as plsc

N = 10000
E = 320000
D = 128
NC = 2
NS = 16
NW = NC * NS
CH = 128
NCH = 80
EP = NCH * CH
NP = NCH // 2
ZR = 80
NZC = N // ZR

_mesh = plsc.VectorSubcoreMesh(core_axis_name="c", subcore_axis_name="s")


def _lane_bcast(v16, j):
    return lax.gather(
        v16, jnp.full((16, 1), j, jnp.int32),
        dimension_numbers=lax.GatherDimensionNumbers(
            offset_dims=(), collapsed_slice_dims=(0,), start_index_map=(0,)),
        slice_sizes=(1,),
        mode=lax.GatherScatterMode.PROMISE_IN_BOUNDS)


@functools.partial(
    pl.kernel,
    out_type=jax.ShapeDtypeStruct((NC, N, D), jnp.float32),
    mesh=_mesh,
    scratch_types=[
        pltpu.VMEM((CH, D), jnp.float32),
        pltpu.VMEM((CH, D), jnp.float32),
        pltpu.VMEM((CH,), jnp.int32),
        pltpu.VMEM((CH,), jnp.int32),
        pltpu.VMEM((CH,), jnp.int32),
        pltpu.VMEM((CH,), jnp.int32),
        pltpu.VMEM((CH,), jnp.float32),
        pltpu.VMEM((CH,), jnp.float32),
        pltpu.SemaphoreType.DMA,
        pltpu.SemaphoreType.DMA,
        pltpu.SemaphoreType.DMA,
        pltpu.SemaphoreType.DMA,
        pltpu.SemaphoreType.DMA,
        pltpu.SemaphoreType.DMA,
        pltpu.SemaphoreType.DMA,
        pltpu.SemaphoreType.DMA,
        pltpu.SemaphoreType.DMA,
        pltpu.SemaphoreType.DMA,
        pltpu.VMEM_SHARED((N, D), jnp.float32),
    ],
)
def _propagate(x_hbm, src_hbm, dst_hbm, w_hbm, out_hbm,
               rows0, rows1, srcb0, srcb1, dstb0, dstb1, wb0, wb1,
               is0, is1, id0, id1, iw0, iw1, g0, g1, s0, s1, acc_sh):
    cid = lax.axis_index("c")
    sid = lax.axis_index("s")
    wid = cid * NS + sid

    zeros16 = jnp.zeros((16,), jnp.float32)
    my_chunks = jnp.where(sid < NZC % NS, NZC // NS + 1, NZC // NS)

    def zfill(i, carry):
        for cc in range(D // 16):
            rows0[i, pl.ds(cc * 16, 16)] = zeros16
        return carry

    lax.fori_loop(0, ZR, zfill, 0)

    def zcopy(k, carry):
        r0 = pl.multiple_of((sid + k * NS) * ZR, 8)
        pltpu.sync_copy(rows0.at[pl.ds(0, ZR)], acc_sh.at[pl.ds(r0, ZR)])
        return carry

    lax.fori_loop(0, my_chunks, zcopy, 0)
    plsc.subcore_barrier()

    def eslice(c, hbm):
        return hbm.at[pl.ds(pl.multiple_of(wid * EP + c * CH, 8), CH)]

    def start_load(c, hbm, buf, sem):
        pltpu.async_copy(eslice(c, hbm), buf, sem)

    def wait_load(c, hbm, buf, sem):
        pltpu.make_async_copy(eslice(c, hbm), buf, sem).wait()

    def start_gather(rows, srcb, sem):
        pltpu.async_copy(x_hbm.at[srcb], rows, sem)

    def wait_gather(rows, srcb, sem):
        pltpu.make_async_copy(x_hbm.at[srcb], rows, sem).wait()

    def start_scatter(rows, dstb, sem):
        pltpu.async_copy(rows, acc_sh.at[dstb], sem, add=True)

    def wait_scatter(rows, dstb, sem):
        pltpu.make_async_copy(rows, acc_sh.at[dstb], sem).wait()

    def scale(rows, wb):
        def group_body(g, gcarry):
            w16 = wb[pl.ds(g * 16, 16)]
            for j in range(16):
                wspl = _lane_bcast(w16, j)
                e = g * 16 + j
                for cc in range(D // 16):
                    sl = pl.ds(cc * 16, 16)
                    rows[e, sl] = rows[e, sl] * wspl
            return gcarry

        lax.fori_loop(0, CH // 16, group_body, 0)

    start_load(0, src_hbm, srcb0, is0)
    start_load(1, src_hbm, srcb1, is1)
    start_load(0, dst_hbm, dstb0, id0)
    start_load(0, w_hbm, wb0, iw0)
    start_load(1, w_hbm, wb1, iw1)
    wait_load(0, src_hbm, srcb0, is0)
    start_gather(rows0, srcb0, g0)

    def pair_body(p, carry):
        c0 = 2 * p
        c1 = c0 + 1
        c2 = jnp.minimum(c0 + 2, NCH - 1)
        c3 = jnp.minimum(c0 + 3, NCH - 1)

        @pl.when(p > 0)
        def _():
            wait_scatter(rows1, dstb1, s1)
        start_load(c1, dst_hbm, dstb1, id1)
        wait_load(c1, src_hbm, srcb1, is1)
        start_gather(rows1, srcb1, g1)
        wait_gather(rows0, srcb0, g0)
        start_load(c2, src_hbm, srcb0, is0)
        wait_load(c0, w_hbm, wb0, iw0)
        scale(rows0, wb0)
        start_load(c2, w_hbm, wb0, iw0)
        wait_load(c0, dst_hbm, dstb0, id0)
        start_scatter(rows0, dstb0, s0)
        wait_gather(rows1, srcb1, g1)
        start_load(c3, src_hbm, srcb1, is1)
        wait_load(c1, w_hbm, wb1, iw1)
        scale(rows1, wb1)
        start_load(c3, w_hbm, wb1, iw1)
        wait_load(c1, dst_hbm, dstb1, id1)
        start_scatter(rows1, dstb1, s1)
        wait_scatter(rows0, dstb0, s0)
        start_load(c2, dst_hbm, dstb0, id0)
        wait_load(c2, src_hbm, srcb0, is0)
        start_gather(rows0, srcb0, g0)
        return carry

    lax.fori_loop(0, NP, pair_body, 0)
    last = NCH - 1
    wait_load(last, src_hbm, srcb1, is1)
    wait_load(last, w_hbm, wb0, iw0)
    wait_load(last, w_hbm, wb1, iw1)
    wait_load(last, dst_hbm, dstb0, id0)
    wait_gather(rows0, srcb0, g0)
    wait_scatter(rows1, dstb1, s1)
    plsc.subcore_barrier()

    def ocopy(k, carry):
        r0 = pl.multiple_of((sid + k * NS) * ZR, 8)
        pltpu.sync_copy(acc_sh.at[pl.ds(r0, ZR)],
                        out_hbm.at[cid, pl.ds(r0, ZR)])
        return carry

    lax.fori_loop(0, my_chunks, ocopy, 0)


_BM = 2000


def _mm_body(hp_ref, w_ref, o_ref):
    h = hp_ref[0] + hp_ref[1]
    o_ref[...] = jnp.dot(h, w_ref[...], preferred_element_type=jnp.float32)


def _matmul(hp, W0):
    return pl.pallas_call(
        _mm_body,
        grid=(N // _BM,),
        in_specs=[
            pl.BlockSpec((NC, _BM, D), lambda i: (0, i, 0)),
            pl.BlockSpec((D, D), lambda i: (0, 0)),
        ],
        out_specs=pl.BlockSpec((_BM, D), lambda i: (i, 0)),
        out_shape=jax.ShapeDtypeStruct((N, D), jnp.float32),
    )(hp, W0)


def kernel(x, edge_index, edge_weight, W0):
    pad = NW * EP - E
    zi = jnp.zeros((pad,), jnp.int32)
    dst = jnp.concatenate([edge_index[0].astype(jnp.int32), zi])
    src = jnp.concatenate([edge_index[1].astype(jnp.int32), zi])
    w = jnp.concatenate([edge_weight.astype(jnp.float32),
                         jnp.zeros((pad,), jnp.float32)])
    hp = _propagate(x, src, dst, w)
    return _matmul(hp, W0)


# speedup vs baseline: 1.0591x; 1.0591x over previous
import functools

import jax
import jax.numpy as jnp
from jax import lax
from jax.experimental import pallas as pl
from jax.experimental.pallas import tpu as pltpu
from jax.experimental.pallas import tpu_sc as plsc

N = 10000
E = 320000
D = 128
NC = 2
NS = 16
NW = NC * NS
CH = 80
NCH = 128
EP = NCH * CH
NB = 4
ZR = 80
NZC = N // ZR

_mesh = plsc.VectorSubcoreMesh(core_axis_name="c", subcore_axis_name="s")


def _lane_bcast(v16, j):
    return lax.gather(
        v16, jnp.full((16, 1), j, jnp.int32),
        dimension_numbers=lax.GatherDimensionNumbers(
            offset_dims=(), collapsed_slice_dims=(0,), start_index_map=(0,)),
        slice_sizes=(1,),
        mode=lax.GatherScatterMode.PROMISE_IN_BOUNDS)


_SCRATCH = (
    [pltpu.VMEM((CH, D), jnp.float32)] * NB
    + [pltpu.VMEM((CH,), jnp.int32)] * NB
    + [pltpu.VMEM((CH,), jnp.int32)] * NB
    + [pltpu.VMEM((CH,), jnp.float32)] * NB
    + [pltpu.SemaphoreType.DMA] * (5 * NB)
    + [pltpu.VMEM_SHARED((N, D), jnp.float32)]
)


@functools.partial(
    pl.kernel,
    out_type=jax.ShapeDtypeStruct((NC, N, D), jnp.float32),
    mesh=_mesh,
    scratch_types=_SCRATCH,
)
def _propagate(x_hbm, src_hbm, dst_hbm, w_hbm, out_hbm, *sc):
    rows = sc[0:NB]
    srcb = sc[NB:2 * NB]
    dstb = sc[2 * NB:3 * NB]
    wb = sc[3 * NB:4 * NB]
    gs = sc[4 * NB:5 * NB]
    ss = sc[5 * NB:6 * NB]
    ls = sc[6 * NB:7 * NB]
    ld = sc[7 * NB:8 * NB]
    lw = sc[8 * NB:9 * NB]
    acc_sh = sc[9 * NB]

    cid = lax.axis_index("c")
    sid = lax.axis_index("s")
    wid = cid * NS + sid

    zeros16 = jnp.zeros((16,), jnp.float32)
    my_chunks = jnp.where(sid < NZC % NS, NZC // NS + 1, NZC // NS)

    def zfill(i, carry):
        for cc in range(D // 16):
            rows[0][i, pl.ds(cc * 16, 16)] = zeros16
        return carry

    lax.fori_loop(0, ZR, zfill, 0)

    def zcopy(k, carry):
        r0 = pl.multiple_of((sid + k * NS) * ZR, 8)
        pltpu.sync_copy(rows[0], acc_sh.at[pl.ds(r0, ZR)])
        return carry

    lax.fori_loop(0, my_chunks, zcopy, 0)
    plsc.subcore_barrier()

    def eslice(c, hbm):
        return hbm.at[pl.ds(pl.multiple_of(wid * EP + c * CH, 8), CH)]

    def start_load(c, hbm, buf, sem):
        pltpu.async_copy(eslice(c, hbm), buf, sem)

    def wait_load(c, hbm, buf, sem):
        pltpu.make_async_copy(eslice(c, hbm), buf, sem).wait()

    def start_gather(b):
        pltpu.async_copy(x_hbm.at[srcb[b]], rows[b], gs[b])

    def wait_gather(b):
        pltpu.make_async_copy(x_hbm.at[srcb[b]], rows[b], gs[b]).wait()

    def start_scatter(b):
        pltpu.async_copy(rows[b], acc_sh.at[dstb[b]], ss[b], add=True)

    def wait_scatter(b):
        pltpu.make_async_copy(rows[b], acc_sh.at[dstb[b]], ss[b]).wait()

    def scale(b):
        def group_body(g, gcarry):
            w16 = wb[b][pl.ds(g * 16, 16)]
            for j in range(16):
                wspl = _lane_bcast(w16, j)
                e = g * 16 + j
                for cc in range(D // 16):
                    sl = pl.ds(cc * 16, 16)
                    rows[b][e, sl] = rows[b][e, sl] * wspl
            return gcarry

        lax.fori_loop(0, CH // 16, group_body, 0)

    for b in range(NB):
        start_load(b, src_hbm, srcb[b], ls[b])
        start_load(b, w_hbm, wb[b], lw[b])
    for b in range(2):
        start_load(b, dst_hbm, dstb[b], ld[b])
    for b in range(2):
        wait_load(b, src_hbm, srcb[b], ls[b])
        start_gather(b)

    def round_body(r, carry):
        for b in range(NB):
            c = 4 * r + b
            c2 = jnp.minimum(c + 2, NCH - 1)
            c4 = jnp.minimum(c + 4, NCH - 1)
            b2 = (b + 2) % NB

            wait_gather(b)
            start_load(c4, src_hbm, srcb[b], ls[b])
            wait_load(c, w_hbm, wb[b], lw[b])
            scale(b)
            start_load(c4, w_hbm, wb[b], lw[b])
            wait_load(c, dst_hbm, dstb[b], ld[b])
            start_scatter(b)
            if b >= 2:
                wait_scatter(b2)
            else:
                @pl.when(r > 0)
                def _():
                    wait_scatter(b2)
            start_load(c2, dst_hbm, dstb[b2], ld[b2])
            wait_load(c2, src_hbm, srcb[b2], ls[b2])
            start_gather(b2)
        return carry

    lax.fori_loop(0, NCH // NB, round_body, 0)

    last = NCH - 1
    for b in range(2):
        wait_load(last, src_hbm, srcb[b + 2], ls[b + 2])
        wait_load(last, dst_hbm, dstb[b], ld[b])
        wait_gather(b)
        wait_scatter(b + 2)
    for b in range(NB):
        wait_load(last, w_hbm, wb[b], lw[b])
    plsc.subcore_barrier()

    def ocopy(k, carry):
        r0 = pl.multiple_of((sid + k * NS) * ZR, 8)
        pltpu.sync_copy(acc_sh.at[pl.ds(r0, ZR)],
                        out_hbm.at[cid, pl.ds(r0, ZR)])
        return carry

    lax.fori_loop(0, my_chunks, ocopy, 0)


_BM = 2000


def _mm_body(hp_ref, w_ref, o_ref):
    h = hp_ref[0] + hp_ref[1]
    o_ref[...] = jnp.dot(h, w_ref[...], preferred_element_type=jnp.float32)


def _matmul(hp, W0):
    return pl.pallas_call(
        _mm_body,
        grid=(N // _BM,),
        in_specs=[
            pl.BlockSpec((NC, _BM, D), lambda i: (0, i, 0)),
            pl.BlockSpec((D, D), lambda i: (0, 0)),
        ],
        out_specs=pl.BlockSpec((_BM, D), lambda i: (i, 0)),
        out_shape=jax.ShapeDtypeStruct((N, D), jnp.float32),
    )(hp, W0)


def kernel(x, edge_index, edge_weight, W0):
    pad = NW * EP - E
    zi = jnp.zeros((pad,), jnp.int32)
    dst = jnp.concatenate([edge_index[0].astype(jnp.int32), zi])
    src = jnp.concatenate([edge_index[1].astype(jnp.int32), zi])
    w = jnp.concatenate([edge_weight.astype(jnp.float32),
                         jnp.zeros((pad,), jnp.float32)])
    hp = _propagate(x, src, dst, w)
    return _matmul(hp, W0)
